# baseline (speedup 1.0000x reference)
import jax
import jax.numpy as jnp
from jax import lax
from jax.experimental import pallas as pl
from jax.experimental.pallas import tpu as pltpu
from jax.experimental.pallas import tpu_sc as plsc

_B = 4096
_L = 200
_DIM = 50
_DOUT = 64
_TPAD = 128
_N = _B * _L

_NC = 2
_NS = 16
_NW = _NC * _NS

_IW = 128
_IROWS = _N // _IW
_IROWS_PER_W = _IROWS // _NW
_TILE = 4
_NCHUNK = _IROWS_PER_W // _TILE
_CHUNK = _TILE * _IW


def _sc_body(idx_hbm, table_hbm, out_hbm, idx_v, rows_v, sem):
    wid = lax.axis_index("s") * _NC + lax.axis_index("c")
    base = wid * _IROWS_PER_W

    def step(i, carry):
        roff = base + i * _TILE
        pltpu.sync_copy(idx_hbm.at[pl.ds(roff, _TILE)], idx_v)
        for j in range(_TILE):
            pltpu.async_copy(
                table_hbm.at[idx_v.at[j]],
                rows_v.at[pl.ds(j * _IW, _IW)],
                sem,
            )
        for j in range(_TILE):
            pltpu.make_async_copy(
                table_hbm.at[idx_v.at[j]],
                rows_v.at[pl.ds(j * _IW, _IW)],
                sem,
            ).wait()
        pltpu.sync_copy(
            rows_v.at[:, pl.ds(0, _DOUT)],
            out_hbm.at[pl.ds(roff * _IW, _CHUNK)],
        )
        return carry

    lax.fori_loop(0, _NCHUNK, step, 0)


@jax.jit
def _sc_gather(idx2d, table_pad):
    fn = pl.kernel(
        _sc_body,
        mesh=plsc.VectorSubcoreMesh(core_axis_name="c", subcore_axis_name="s"),
        out_type=jax.ShapeDtypeStruct((_N, _DOUT), jnp.float32),
        scratch_types=[
            pltpu.VMEM((_TILE, _IW), jnp.int32),
            pltpu.VMEM((_CHUNK, _TPAD), jnp.float32),
            pltpu.SemaphoreType.DMA,
        ],
        compiler_params=pltpu.CompilerParams(use_tc_tiling_on_sc=False),
    )
    return fn(idx2d, table_pad)


def kernel(indices, table):
    idx2d = indices.reshape(_IROWS, _IW).astype(jnp.int32)
    table_pad = jnp.pad(table, ((0, 0), (0, _TPAD - _DIM)))
    out = _sc_gather(idx2d, table_pad)
    return out[:, :_DIM].reshape(_B, _L, _DIM)


# speedup vs baseline: 1.8446x; 1.8446x over previous
import jax
import jax.numpy as jnp
from jax import lax
from jax.experimental import pallas as pl
from jax.experimental.pallas import tpu as pltpu
from jax.experimental.pallas import tpu_sc as plsc

_B = 4096
_L = 200
_DIM = 50
_DGATH = 64
_DOUT = 128
_TPAD = 128
_N = _B * _L

_NC = 2
_NS = 16
_NW = _NC * _NS

_IW = 128
_IROWS = _N // _IW
_IROWS_PER_W = _IROWS // _NW
_TILE = 8
_NCHUNK = _IROWS_PER_W // _TILE
_CHUNK = _TILE * _IW


def _sc_body(idx_hbm, table_hbm, out_hbm, idx_v, rows_v, sem):
    wid = lax.axis_index("s") * _NC + lax.axis_index("c")
    base = wid * _IROWS_PER_W

    def step(i, carry):
        roff = base + i * _TILE
        pltpu.sync_copy(idx_hbm.at[pl.ds(roff, _TILE)], idx_v)
        for j in range(_TILE):
            pltpu.async_copy(
                table_hbm.at[idx_v.at[j]],
                rows_v.at[pl.ds(j * _IW, _IW)],
                sem,
            )
        for j in range(_TILE):
            pltpu.make_async_copy(
                table_hbm.at[idx_v.at[j]],
                rows_v.at[pl.ds(j * _IW, _IW)],
                sem,
            ).wait()
        pltpu.sync_copy(
            rows_v,
            out_hbm.at[pl.ds(roff * _IW, _CHUNK), pl.ds(0, _DGATH)],
        )
        return carry

    lax.fori_loop(0, _NCHUNK, step, 0)


@jax.jit
def _sc_gather(idx2d, table_pad2):
    fn = pl.kernel(
        _sc_body,
        mesh=plsc.VectorSubcoreMesh(core_axis_name="c", subcore_axis_name="s"),
        out_type=jax.ShapeDtypeStruct((_N, _DOUT), jnp.float32),
        scratch_types=[
            pltpu.VMEM((_TILE, _IW), jnp.int32),
            pltpu.VMEM((_CHUNK, _DGATH), jnp.float32),
            pltpu.SemaphoreType.DMA,
        ],
        compiler_params=pltpu.CompilerParams(use_tc_tiling_on_sc=False),
    )
    return fn(idx2d, table_pad2)


def kernel(indices, table):
    idx2d = indices.reshape(_IROWS, _IW).astype(jnp.int32) * 2
    table_pad2 = jnp.pad(table, ((0, 0), (0, _TPAD - _DIM))).reshape(
        2 * 400000, _DGATH
    )
    out = _sc_gather(idx2d, table_pad2)
    return out[:, :_DIM].reshape(_B, _L, _DIM)


# speedup vs baseline: 1.9123x; 1.0367x over previous
import jax
import jax.numpy as jnp
from jax import lax
from jax.experimental import pallas as pl
from jax.experimental.pallas import tpu as pltpu
from jax.experimental.pallas import tpu_sc as plsc

_B = 4096
_L = 200
_DIM = 50
_DGATH = 64
_DOUT = 128
_TPAD = 128
_N = _B * _L

_NC = 2
_NS = 16
_NW = _NC * _NS

_IW = 128
_IROWS = _N // _IW
_IROWS_PER_W = _IROWS // _NW
_TILE = 4
_NCHUNK = _IROWS_PER_W // _TILE
_NE = _NCHUNK // 2
_CHUNK = _TILE * _IW


def _sc_body(idx_hbm, table_hbm, out_hbm,
             idx0, idx1, rows0, rows1,
             s_i0, s_i1, s_g0, s_g1, s_o0, s_o1):
    wid = lax.axis_index("s") * _NC + lax.axis_index("c")
    base = wid * _IROWS_PER_W
    idx_bufs = (idx0, idx1)
    row_bufs = (rows0, rows1)
    s_i = (s_i0, s_i1)
    s_g = (s_g0, s_g1)
    s_o = (s_o0, s_o1)

    def idx_copy(i, b, s):
        return pltpu.make_async_copy(
            idx_hbm.at[pl.ds(base + i * _TILE, _TILE)], idx_bufs[b], s
        )

    def out_copy(i, b, s):
        return pltpu.make_async_copy(
            row_bufs[b],
            out_hbm.at[pl.ds((base + i * _TILE) * _IW, _CHUNK),
                       pl.ds(0, _DGATH)],
            s,
        )

    idx_copy(0, 0, s_i[0]).start()

    def epoch(e, carry):
        for b in range(2):
            i = 2 * e + b
            nxt = i + 1
            @pl.when(nxt < _NCHUNK)
            def _():
                idx_copy(nxt, 1 - b, s_i[1 - b]).start()

            idx_copy(i, b, s_i[b]).wait()
            @pl.when(e > 0)
            def _():
                out_copy(i - 2, b, s_o[b]).wait()

            for j in range(_TILE):
                pltpu.async_copy(
                    table_hbm.at[idx_bufs[b].at[j]],
                    row_bufs[b].at[pl.ds(j * _IW, _IW)],
                    s_g[b],
                )
            for j in range(_TILE):
                pltpu.make_async_copy(
                    table_hbm.at[idx_bufs[b].at[j]],
                    row_bufs[b].at[pl.ds(j * _IW, _IW)],
                    s_g[b],
                ).wait()
            out_copy(i, b, s_o[b]).start()
        return carry

    lax.fori_loop(0, _NE, epoch, 0)
    out_copy(_NCHUNK - 2, 0, s_o[0]).wait()
    out_copy(_NCHUNK - 1, 1, s_o[1]).wait()


@jax.jit
def _sc_gather(idx2d, table_pad2):
    fn = pl.kernel(
        _sc_body,
        mesh=plsc.VectorSubcoreMesh(core_axis_name="c", subcore_axis_name="s"),
        out_type=jax.ShapeDtypeStruct((_N, _DOUT), jnp.float32),
        scratch_types=[
            pltpu.VMEM((_TILE, _IW), jnp.int32),
            pltpu.VMEM((_TILE, _IW), jnp.int32),
            pltpu.VMEM((_CHUNK, _DGATH), jnp.float32),
            pltpu.VMEM((_CHUNK, _DGATH), jnp.float32),
            pltpu.SemaphoreType.DMA,
            pltpu.SemaphoreType.DMA,
            pltpu.SemaphoreType.DMA,
            pltpu.SemaphoreType.DMA,
            pltpu.SemaphoreType.DMA,
            pltpu.SemaphoreType.DMA,
        ],
        compiler_params=pltpu.CompilerParams(use_tc_tiling_on_sc=False),
    )
    return fn(idx2d, table_pad2)


def kernel(indices, table):
    idx2d = indices.reshape(_IROWS, _IW).astype(jnp.int32) * 2
    table_pad2 = jnp.pad(table, ((0, 0), (0, _TPAD - _DIM))).reshape(
        2 * 400000, _DGATH
    )
    out = _sc_gather(idx2d, table_pad2)
    return out[:, :_DIM].reshape(_B, _L, _DIM)
